# baseline (speedup 1.0000x reference)
import functools

import jax
import jax.numpy as jnp
from jax import lax
from jax.experimental import pallas as pl
from jax.experimental.pallas import tpu as pltpu
from jax.experimental.pallas import tpu_sc as plsc

N = 4
T = 4096
U = 1024
SCALE_F = float(U) ** 0.5

TB = 1024
NC, NS = 2, 16
NW = NC * NS
ROWS_PER_W = T // NW
CHUNK = 32


_HALF_PI = 1.5707963267948966


_SUB = 16
_NHI = TB // _SUB
_NSTEPS = T // TB


def _table_body(o_ref, s1_ref, c1_ref, sb_ref, cb_ref):
    i = pl.program_id(0)

    @pl.when(i == 0)
    def _():
        half_pi = jnp.float32(_HALF_PI)

        colf = lax.broadcasted_iota(jnp.int32, (_SUB, U), 1).astype(jnp.float32)
        w_lo = jnp.power(jnp.float32(10000.0), colf * jnp.float32(-2.0 / U))
        rows_lo = lax.broadcasted_iota(jnp.int32, (_SUB, U), 0).astype(jnp.float32)
        parity_odd = (lax.broadcasted_iota(jnp.int32, (_SUB, U), 1) % 2) == 1
        a_lo = rows_lo * w_lo + jnp.where(parity_odd, half_pi, jnp.float32(0.0))
        s_lo = jnp.sin(a_lo)
        c_lo = jnp.sin(a_lo + half_pi)

        colh = lax.broadcasted_iota(jnp.int32, (_NHI, U), 1).astype(jnp.float32)
        w_hi = jnp.power(jnp.float32(10000.0), colh * jnp.float32(-2.0 / U))
        rows_hi = lax.broadcasted_iota(jnp.int32, (_NHI, U), 0).astype(jnp.float32)
        a_hi = rows_hi * (jnp.float32(_SUB) * w_hi)
        s_hi = jnp.sin(a_hi)
        c_hi = jnp.sin(a_hi + half_pi)

        cols = lax.broadcasted_iota(jnp.int32, (_NSTEPS, U), 1).astype(jnp.float32)
        w_s = jnp.power(jnp.float32(10000.0), cols * jnp.float32(-2.0 / U))
        rows_s = lax.broadcasted_iota(jnp.int32, (_NSTEPS, U), 0).astype(jnp.float32)
        b_all = rows_s * (jnp.float32(TB) * w_s)
        sb_ref[...] = jnp.sin(b_all)
        cb_ref[...] = jnp.sin(b_all + half_pi)

        scale = jnp.float32(SCALE_F)
        for rh in range(_NHI):
            sh = s_hi[rh : rh + 1, :] * scale
            ch = c_hi[rh : rh + 1, :] * scale
            s1_ref[rh * _SUB : (rh + 1) * _SUB, :] = s_lo * ch + c_lo * sh
            c1_ref[rh * _SUB : (rh + 1) * _SUB, :] = c_lo * ch - s_lo * sh

    sb = sb_ref[pl.ds(i, 1), :]
    cb = cb_ref[pl.ds(i, 1), :]
    o_ref[...] = s1_ref[...] * cb + c1_ref[...] * sb

    @pl.when(i == 0)
    def _():
        o_ref[0:1, :] = jnp.zeros((1, U), jnp.float32)


def _make_table():
    return pl.pallas_call(
        _table_body,
        grid=(_NSTEPS,),
        out_specs=pl.BlockSpec((TB, U), lambda i: (i, 0)),
        out_shape=jax.ShapeDtypeStruct((T, U), jnp.float32),
        scratch_shapes=[
            pltpu.VMEM((TB, U), jnp.float32),
            pltpu.VMEM((TB, U), jnp.float32),
            pltpu.VMEM((_NSTEPS, U), jnp.float32),
            pltpu.VMEM((_NSTEPS, U), jnp.float32),
        ],
    )()


def _sc_broadcast_body(table_hbm, out_hbm, buf0, buf1, rsem, wsem):
    wid = lax.axis_index("s") * NC + lax.axis_index("c")
    base0 = wid * ROWS_PER_W
    bufs = (buf0, buf1)
    nch = ROWS_PER_W // CHUNK
    whs = [None, None]
    rcp = pltpu.async_copy(table_hbm.at[pl.ds(base0, CHUNK), :], bufs[0], rsem)
    for c in range(nch):
        b = c % 2
        nb = (c + 1) % 2
        base = base0 + c * CHUNK
        nrcp = None
        if c + 1 < nch:
            if whs[nb] is not None:
                for w in whs[nb]:
                    w.wait()
                whs[nb] = None
            nrcp = pltpu.async_copy(
                table_hbm.at[pl.ds(base + CHUNK, CHUNK), :], bufs[nb], rsem)
        rcp.wait()
        whs[b] = [
            pltpu.async_copy(bufs[b], out_hbm.at[pl.ds(n * T + base, CHUNK), :], wsem)
            for n in range(N)
        ]
        rcp = nrcp
    for hs in whs:
        if hs is not None:
            for w in hs:
                w.wait()


@functools.cache
def _sc_broadcast():
    return pl.kernel(
        _sc_broadcast_body,
        out_type=jax.ShapeDtypeStruct((N * T, U), jnp.float32),
        mesh=plsc.VectorSubcoreMesh(core_axis_name="c", subcore_axis_name="s"),
        scratch_types=[
            pltpu.VMEM((CHUNK, U), jnp.float32),
            pltpu.VMEM((CHUNK, U), jnp.float32),
            pltpu.SemaphoreType.DMA,
            pltpu.SemaphoreType.DMA,
        ],
    )


def kernel(inputs):
    del inputs
    table = _make_table()
    flat = _sc_broadcast()(table)
    return flat.reshape(N, T, U)


# speedup vs baseline: 1.0202x; 1.0202x over previous
import functools

import jax
import jax.numpy as jnp
from jax import lax
from jax.experimental import pallas as pl
from jax.experimental.pallas import tpu as pltpu
from jax.experimental.pallas import tpu_sc as plsc

N = 4
T = 4096
U = 1024
SCALE_F = float(U) ** 0.5

TB = 512
NC, NS = 2, 16
NW = NC * NS
ROWS_PER_W = T // NW
CHUNK = 32


_HALF_PI = 1.5707963267948966


_SUB = 16
_NHI = TB // _SUB
_NSTEPS = T // TB


def _table_body(o_ref, s1_ref, c1_ref, sb_ref, cb_ref):
    i = pl.program_id(0)

    @pl.when(i == 0)
    def _():
        half_pi = jnp.float32(_HALF_PI)

        colf = lax.broadcasted_iota(jnp.int32, (_SUB, U), 1).astype(jnp.float32)
        w_lo = jnp.power(jnp.float32(10000.0), colf * jnp.float32(-2.0 / U))
        rows_lo = lax.broadcasted_iota(jnp.int32, (_SUB, U), 0).astype(jnp.float32)
        parity_odd = (lax.broadcasted_iota(jnp.int32, (_SUB, U), 1) % 2) == 1
        a_lo = rows_lo * w_lo + jnp.where(parity_odd, half_pi, jnp.float32(0.0))
        s_lo = jnp.sin(a_lo)
        c_lo = jnp.sin(a_lo + half_pi)

        colh = lax.broadcasted_iota(jnp.int32, (_NHI, U), 1).astype(jnp.float32)
        w_hi = jnp.power(jnp.float32(10000.0), colh * jnp.float32(-2.0 / U))
        rows_hi = lax.broadcasted_iota(jnp.int32, (_NHI, U), 0).astype(jnp.float32)
        a_hi = rows_hi * (jnp.float32(_SUB) * w_hi)
        s_hi = jnp.sin(a_hi)
        c_hi = jnp.sin(a_hi + half_pi)

        cols = lax.broadcasted_iota(jnp.int32, (_NSTEPS, U), 1).astype(jnp.float32)
        w_s = jnp.power(jnp.float32(10000.0), cols * jnp.float32(-2.0 / U))
        rows_s = lax.broadcasted_iota(jnp.int32, (_NSTEPS, U), 0).astype(jnp.float32)
        b_all = rows_s * (jnp.float32(TB) * w_s)
        sb_ref[...] = jnp.sin(b_all)
        cb_ref[...] = jnp.sin(b_all + half_pi)

        scale = jnp.float32(SCALE_F)
        for rh in range(_NHI):
            sh = s_hi[rh : rh + 1, :] * scale
            ch = c_hi[rh : rh + 1, :] * scale
            s1_ref[rh * _SUB : (rh + 1) * _SUB, :] = s_lo * ch + c_lo * sh
            c1_ref[rh * _SUB : (rh + 1) * _SUB, :] = c_lo * ch - s_lo * sh

    sb = sb_ref[pl.ds(i, 1), :]
    cb = cb_ref[pl.ds(i, 1), :]
    o_ref[...] = s1_ref[...] * cb + c1_ref[...] * sb

    @pl.when(i == 0)
    def _():
        o_ref[0:1, :] = jnp.zeros((1, U), jnp.float32)


def _make_table():
    return pl.pallas_call(
        _table_body,
        grid=(_NSTEPS,),
        out_specs=pl.BlockSpec((TB, U), lambda i: (i, 0)),
        out_shape=jax.ShapeDtypeStruct((T, U), jnp.float32),
        scratch_shapes=[
            pltpu.VMEM((TB, U), jnp.float32),
            pltpu.VMEM((TB, U), jnp.float32),
            pltpu.VMEM((_NSTEPS, U), jnp.float32),
            pltpu.VMEM((_NSTEPS, U), jnp.float32),
        ],
    )()


_NBUF = 3


def _sc_broadcast_body(table_hbm, out_hbm, buf0, buf1, buf2, rsem, wsem):
    wid = lax.axis_index("s") * NC + lax.axis_index("c")
    base0 = wid * ROWS_PER_W
    bufs = (buf0, buf1, buf2)
    nch = ROWS_PER_W // CHUNK
    whs = [None] * _NBUF
    rcp = pltpu.async_copy(table_hbm.at[pl.ds(base0, CHUNK), :], bufs[0], rsem)
    for c in range(nch):
        b = c % _NBUF
        nb = (c + 1) % _NBUF
        base = base0 + c * CHUNK
        nrcp = None
        if c + 1 < nch:
            if whs[nb] is not None:
                for w in whs[nb]:
                    w.wait()
                whs[nb] = None
            nrcp = pltpu.async_copy(
                table_hbm.at[pl.ds(base + CHUNK, CHUNK), :], bufs[nb], rsem)
        rcp.wait()
        whs[b] = [
            pltpu.async_copy(bufs[b], out_hbm.at[pl.ds(n * T + base, CHUNK), :], wsem)
            for n in range(N)
        ]
        rcp = nrcp
    for hs in whs:
        if hs is not None:
            for w in hs:
                w.wait()


@functools.cache
def _sc_broadcast():
    return pl.kernel(
        _sc_broadcast_body,
        out_type=jax.ShapeDtypeStruct((N * T, U), jnp.float32),
        mesh=plsc.VectorSubcoreMesh(core_axis_name="c", subcore_axis_name="s"),
        scratch_types=[
            pltpu.VMEM((CHUNK, U), jnp.float32),
            pltpu.VMEM((CHUNK, U), jnp.float32),
            pltpu.VMEM((CHUNK, U), jnp.float32),
            pltpu.SemaphoreType.DMA,
            pltpu.SemaphoreType.DMA,
        ],
    )


def kernel(inputs):
    del inputs
    table = _make_table()
    flat = _sc_broadcast()(table)
    return flat.reshape(N, T, U)
